# baseline (speedup 1.0000x reference)
import functools

import jax
import jax.numpy as jnp
from jax import lax
from jax.experimental import pallas as pl
from jax.experimental.pallas import tpu as pltpu
from jax.experimental.pallas import tpu_sc as plsc

INPUT_SIZE = 768.0
CONF_T = 0.01
IOU_T = 0.45
NPRED = 10

B = 8
N = 20000
C = 22
SPB = 4
NLOC = N // SPB
CHUNK = 200
NCHUNKS = NLOC // CHUNK
GPC = (CHUNK + 15) // 16
PAD = 5120
NGROUPS = PAD // 16
S = PAD
L = 16


def _splat_i(k):
    return jnp.full((L,), k, jnp.int32)


def _body(y, out, stage0, stage1, soa, g64, rec16, postbuf, recbuf, groupbuf,
          shared, sem0, sem1):
    c_id = lax.axis_index("c")
    s_id = lax.axis_index("s")
    b = c_id * SPB + s_id // SPB
    sl = s_id % SPB
    base = sl * NLOC
    leader = sl == 0
    iot = lax.iota(jnp.int32, L)

    stages = (stage0, stage1)
    sems = (sem0, sem1)
    copies = [
        pltpu.make_async_copy(
            y.at[b, pl.ds(base + c * CHUNK, CHUNK), :],
            stages[c % 2], sems[c % 2])
        for c in range(NCHUNKS)
    ]
    copies[0].start()
    for c in range(NCHUNKS):
        if c + 1 < NCHUNKS:
            copies[c + 1].start()
        copies[c].wait()
        stage = stages[c % 2]
        lch = c * CHUNK

        def dec_body(g, carry, stage=stage, lch=lch):
            rows = jnp.minimum(g * L + iot, CHUNK - 1)

            def ld(k):
                return plsc.load_gather(stage, [rows, _splat_i(k)])

            off = lch + g * L
            sc = ld(1)
            sc = jnp.where(sc < CONF_T, -1.0, sc)
            pcx = ld(14)
            pcy = ld(15)
            pw = ld(16) + 1e-3
            ph = ld(17) + 1e-3
            va = ld(18) * pw
            vb = ld(19) * ph
            cx = ld(2) * va + pcx
            cy = ld(3) * vb + pcy
            w = jnp.exp(jnp.clip(ld(4) * ld(20), -10.0, 10.0)) * pw
            h = jnp.exp(jnp.clip(ld(5) * ld(21), -10.0, 10.0)) * ph
            x1 = (cx - 0.5 * w) * INPUT_SIZE
            y1 = (cy - 0.5 * h) * INPUT_SIZE
            x2 = (cx + 0.5 * w) * INPUT_SIZE
            y2 = (cy + 0.5 * h) * INPUT_SIZE
            ar = jnp.maximum(x2 - x1, 0.0) * jnp.maximum(y2 - y1, 0.0)
            soa[pl.ds(off, L)] = sc
            soa[pl.ds(S + off, L)] = x1
            soa[pl.ds(2 * S + off, L)] = y1
            soa[pl.ds(3 * S + off, L)] = x2
            soa[pl.ds(4 * S + off, L)] = y2
            soa[pl.ds(5 * S + off, L)] = ar
            for i in range(4):
                qx = (ld(6 + 2 * i) * va + pcx) * INPUT_SIZE
                qy = (ld(7 + 2 * i) * vb + pcy) * INPUT_SIZE
                soa[pl.ds((6 + 2 * i) * S + off, L)] = qx
                soa[pl.ds((7 + 2 * i) * S + off, L)] = qy
            return carry

        lax.fori_loop(0, GPC, dec_body, 0)

    pad0 = (NLOC // L) * L

    def pad_body(g, carry):
        off = pad0 + g * L
        m = (off + iot) >= NLOC
        sv = soa[pl.ds(off, L)]
        soa[pl.ds(off, L)] = jnp.where(m, -1.0, sv)
        for r in range(1, 6):
            v = soa[pl.ds(r * S + off, L)]
            soa[pl.ds(r * S + off, L)] = jnp.where(m, 0.0, v)
        return carry

    lax.fori_loop(0, (PAD - pad0) // L, pad_body, 0)

    def step_body(step, carry):
        bx1, by1, bx2, by2, barea, wlidx = carry

        def pass_body(g, vc):
            vbest, vbidx = vc
            o = g * L
            sv = soa[pl.ds(o, L)]
            x1 = soa[pl.ds(S + o, L)]
            y1 = soa[pl.ds(2 * S + o, L)]
            x2 = soa[pl.ds(3 * S + o, L)]
            y2 = soa[pl.ds(4 * S + o, L)]
            ar = soa[pl.ds(5 * S + o, L)]
            iw = jnp.maximum(jnp.minimum(bx2, x2) - jnp.maximum(bx1, x1), 0.0)
            ih = jnp.maximum(jnp.minimum(by2, y2) - jnp.maximum(by1, y1), 0.0)
            inter = iw * ih
            iou = inter / (barea + ar - inter + 1e-8)
            lidx = o + iot
            supp = (iou > IOU_T) | (lidx == wlidx)
            s2 = jnp.where(supp, -1.0, sv)
            soa[pl.ds(o, L)] = s2
            upd = s2 > vbest
            vbest = jnp.where(upd, s2, vbest)
            vbidx = jnp.where(upd, lidx, vbidx)
            return (vbest, vbidx)

        vbest, vbidx = lax.fori_loop(
            0, NGROUPS, pass_body,
            (jnp.full((L,), -2.0, jnp.float32), jnp.zeros((L,), jnp.int32)))

        m = jnp.max(vbest)
        lidxm = jnp.where(vbest == m, vbidx, jnp.int32(2 ** 30))
        lidx = jnp.min(lidxm)
        gidxf = (base + lidx).astype(jnp.float32)

        rows = jnp.minimum(iot + jnp.where(iot >= 5, 1, 0), 13)
        cand = plsc.load_gather(soa, [rows * S + lidx])
        cand = jnp.where(iot == 13, gidxf, cand)
        cand = jnp.where(iot >= 14, 0.0, cand)
        postbuf[0, 0:L] = cand
        pltpu.sync_copy(postbuf, shared.at[s_id])
        plsc.subcore_barrier()
        pltpu.sync_copy(shared.at[pl.ds((s_id // SPB) * SPB, SPB), :, :],
                        groupbuf)
        plsc.subcore_barrier()
        for j in range(SPB):
            g64[pl.ds(j * L, L)] = groupbuf[j, 0, 0:L]

        rsel = jnp.minimum(iot, SPB - 1)
        sc4 = plsc.load_gather(g64, [rsel * L])
        id4 = plsc.load_gather(g64, [rsel * L + 13])
        lanem = iot < SPB
        sc4 = jnp.where(lanem, sc4, -3.0)
        m2 = jnp.max(sc4)
        sel2 = (sc4 == m2) & lanem
        wg = jnp.min(jnp.where(sel2, id4, 3e9))
        rowsel = jnp.where(sel2 & (id4 == wg), iot, jnp.int32(99))
        r = jnp.min(rowsel)
        rec = plsc.load_gather(g64, [_splat_i(0) + r * L + iot])
        rec16[...] = rec
        recbuf[0, 0:L] = rec

        @pl.when(leader)
        def _():
            pltpu.sync_copy(recbuf, out.at[b, step])

        nbx1 = plsc.load_gather(rec16, [_splat_i(1)])
        nby1 = plsc.load_gather(rec16, [_splat_i(2)])
        nbx2 = plsc.load_gather(rec16, [_splat_i(3)])
        nby2 = plsc.load_gather(rec16, [_splat_i(4)])
        nwg = plsc.load_gather(rec16, [_splat_i(13)])
        nbarea = jnp.maximum(nbx2 - nbx1, 0.0) * jnp.maximum(nby2 - nby1, 0.0)
        nwlidx = (nwg - base.astype(jnp.float32)).astype(jnp.int32)
        return (nbx1, nby1, nbx2, nby2, nbarea, nwlidx)

    z = jnp.zeros((L,), jnp.float32)
    lax.fori_loop(0, NPRED, step_body,
                  (z, z, z, z, z, jnp.full((L,), -1, jnp.int32)))


@functools.partial(
    pl.kernel,
    out_type=jax.ShapeDtypeStruct((B, NPRED, 8, 128), jnp.float32),
    mesh=plsc.VectorSubcoreMesh(core_axis_name="c", subcore_axis_name="s"),
    compiler_params=pltpu.CompilerParams(
        needs_layout_passes=False, use_tc_tiling_on_sc=True),
    scratch_types=[
        pltpu.VMEM((CHUNK, C), jnp.float32),
        pltpu.VMEM((CHUNK, C), jnp.float32),
        pltpu.VMEM((14 * S,), jnp.float32),
        pltpu.VMEM((SPB * L,), jnp.float32),
        pltpu.VMEM((L,), jnp.float32),
        pltpu.VMEM((8, 128), jnp.float32),
        pltpu.VMEM((8, 128), jnp.float32),
        pltpu.VMEM((SPB, 8, 128), jnp.float32),
        pltpu.VMEM_SHARED((16, 8, 128), jnp.float32),
        pltpu.SemaphoreType.DMA,
        pltpu.SemaphoreType.DMA,
    ],
)
def _sc_nms(y, out, *scratch):
    _body(y, out, *scratch)


def kernel(y_pred):
    out = _sc_nms(y_pred)
    return out[:, :, 0, :13]


# speedup vs baseline: 1.0322x; 1.0322x over previous
import functools

import jax
import jax.numpy as jnp
from jax import lax
from jax.experimental import pallas as pl
from jax.experimental.pallas import tpu as pltpu
from jax.experimental.pallas import tpu_sc as plsc

INPUT_SIZE = 768.0
CONF_T = 0.01
IOU_T = 0.45
NPRED = 10

B = 8
N = 20000
C = 22
SPB = 4
NLOC = N // SPB
CHUNK = 200
NCHUNKS = NLOC // CHUNK
GPC = (CHUNK + 15) // 16
PAD = 5120
NGROUPS = PAD // 16
S = PAD
L = 16


def _splat_i(k):
    return jnp.full((L,), k, jnp.int32)


def _body(y, out, stage0, stage1, soa, g64, rec16, postbuf, recbuf, groupbuf,
          shared, sem0, sem1):
    c_id = lax.axis_index("c")
    s_id = lax.axis_index("s")
    b = c_id * SPB + s_id // SPB
    sl = s_id % SPB
    base = sl * NLOC
    leader = sl == 0
    iot = lax.iota(jnp.int32, L)

    def start_copy(c, stage, sem):
        pltpu.make_async_copy(
            y.at[b, pl.ds(base + c * CHUNK, CHUNK), :], stage, sem).start()

    def wait_copy(stage, sem):
        pltpu.make_async_copy(
            y.at[b, pl.ds(base, CHUNK), :], stage, sem).wait()

    def decode_chunk(c, stage):
        lch = c * CHUNK

        def dec_body(g, carry):
            rows = jnp.minimum(g * L + iot, CHUNK - 1)

            def ld(k):
                return plsc.load_gather(stage, [rows, _splat_i(k)])

            off = lch + g * L
            sc = ld(1)
            sc = jnp.where(sc < CONF_T, -1.0, sc)
            pcx = ld(14)
            pcy = ld(15)
            pw = ld(16) + 1e-3
            ph = ld(17) + 1e-3
            va = ld(18) * pw
            vb = ld(19) * ph
            cx = ld(2) * va + pcx
            cy = ld(3) * vb + pcy
            w = jnp.exp(jnp.clip(ld(4) * ld(20), -10.0, 10.0)) * pw
            h = jnp.exp(jnp.clip(ld(5) * ld(21), -10.0, 10.0)) * ph
            x1 = (cx - 0.5 * w) * INPUT_SIZE
            y1 = (cy - 0.5 * h) * INPUT_SIZE
            x2 = (cx + 0.5 * w) * INPUT_SIZE
            y2 = (cy + 0.5 * h) * INPUT_SIZE
            ar = jnp.maximum(x2 - x1, 0.0) * jnp.maximum(y2 - y1, 0.0)
            soa[pl.ds(off, L)] = sc
            soa[pl.ds(S + off, L)] = x1
            soa[pl.ds(2 * S + off, L)] = y1
            soa[pl.ds(3 * S + off, L)] = x2
            soa[pl.ds(4 * S + off, L)] = y2
            soa[pl.ds(5 * S + off, L)] = ar
            for i in range(4):
                qx = (ld(6 + 2 * i) * va + pcx) * INPUT_SIZE
                qy = (ld(7 + 2 * i) * vb + pcy) * INPUT_SIZE
                soa[pl.ds((6 + 2 * i) * S + off, L)] = qx
                soa[pl.ds((7 + 2 * i) * S + off, L)] = qy
            return carry

        lax.fori_loop(0, GPC, dec_body, 0)

    start_copy(0, stage0, sem0)
    start_copy(1, stage1, sem1)

    def pair_body(p, carry):
        c0 = p * 2

        wait_copy(stage0, sem0)
        decode_chunk(c0, stage0)

        @pl.when(c0 + 2 < NCHUNKS)
        def _():
            start_copy(c0 + 2, stage0, sem0)

        wait_copy(stage1, sem1)
        decode_chunk(c0 + 1, stage1)

        @pl.when(c0 + 3 < NCHUNKS)
        def _():
            start_copy(c0 + 3, stage1, sem1)

        return carry

    lax.fori_loop(0, NCHUNKS // 2, pair_body, 0)
    if NCHUNKS % 2:
        wait_copy(stage0, sem0)
        decode_chunk(NCHUNKS - 1, stage0)

    pad0 = (NLOC // L) * L

    def pad_body(g, carry):
        off = pad0 + g * L
        m = (off + iot) >= NLOC
        sv = soa[pl.ds(off, L)]
        soa[pl.ds(off, L)] = jnp.where(m, -1.0, sv)
        for r in range(1, 6):
            v = soa[pl.ds(r * S + off, L)]
            soa[pl.ds(r * S + off, L)] = jnp.where(m, 0.0, v)
        return carry

    lax.fori_loop(0, (PAD - pad0) // L, pad_body, 0)

    def step_body(step, carry):
        bx1, by1, bx2, by2, barea, wlidx = carry

        def pass_body(g, vc):
            vbest, vbidx = vc
            o = g * L
            sv = soa[pl.ds(o, L)]
            x1 = soa[pl.ds(S + o, L)]
            y1 = soa[pl.ds(2 * S + o, L)]
            x2 = soa[pl.ds(3 * S + o, L)]
            y2 = soa[pl.ds(4 * S + o, L)]
            ar = soa[pl.ds(5 * S + o, L)]
            iw = jnp.maximum(jnp.minimum(bx2, x2) - jnp.maximum(bx1, x1), 0.0)
            ih = jnp.maximum(jnp.minimum(by2, y2) - jnp.maximum(by1, y1), 0.0)
            inter = iw * ih
            iou = inter / (barea + ar - inter + 1e-8)
            lidx = o + iot
            supp = (iou > IOU_T) | (lidx == wlidx)
            s2 = jnp.where(supp, -1.0, sv)
            soa[pl.ds(o, L)] = s2
            upd = s2 > vbest
            vbest = jnp.where(upd, s2, vbest)
            vbidx = jnp.where(upd, lidx, vbidx)
            return (vbest, vbidx)

        vbest, vbidx = lax.fori_loop(
            0, NGROUPS, pass_body,
            (jnp.full((L,), -2.0, jnp.float32), jnp.zeros((L,), jnp.int32)))

        m = jnp.max(vbest)
        lidxm = jnp.where(vbest == m, vbidx, jnp.int32(2 ** 30))
        lidx = jnp.min(lidxm)
        gidxf = (base + lidx).astype(jnp.float32)

        rows = jnp.minimum(iot + jnp.where(iot >= 5, 1, 0), 13)
        cand = plsc.load_gather(soa, [rows * S + lidx])
        cand = jnp.where(iot == 13, gidxf, cand)
        cand = jnp.where(iot >= 14, 0.0, cand)
        postbuf[0, 0:L] = cand
        pltpu.sync_copy(postbuf, shared.at[s_id])
        plsc.subcore_barrier()
        pltpu.sync_copy(shared.at[pl.ds((s_id // SPB) * SPB, SPB), :, :],
                        groupbuf)
        plsc.subcore_barrier()
        for j in range(SPB):
            g64[pl.ds(j * L, L)] = groupbuf[j, 0, 0:L]

        rsel = jnp.minimum(iot, SPB - 1)
        sc4 = plsc.load_gather(g64, [rsel * L])
        id4 = plsc.load_gather(g64, [rsel * L + 13])
        lanem = iot < SPB
        sc4 = jnp.where(lanem, sc4, -3.0)
        m2 = jnp.max(sc4)
        sel2 = (sc4 == m2) & lanem
        wg = jnp.min(jnp.where(sel2, id4, 3e9))
        rowsel = jnp.where(sel2 & (id4 == wg), iot, jnp.int32(99))
        r = jnp.min(rowsel)
        rec = plsc.load_gather(g64, [_splat_i(0) + r * L + iot])
        rec16[...] = rec
        recbuf[0, 0:L] = rec

        @pl.when(leader)
        def _():
            pltpu.sync_copy(recbuf, out.at[b, step])

        nbx1 = plsc.load_gather(rec16, [_splat_i(1)])
        nby1 = plsc.load_gather(rec16, [_splat_i(2)])
        nbx2 = plsc.load_gather(rec16, [_splat_i(3)])
        nby2 = plsc.load_gather(rec16, [_splat_i(4)])
        nwg = plsc.load_gather(rec16, [_splat_i(13)])
        nbarea = jnp.maximum(nbx2 - nbx1, 0.0) * jnp.maximum(nby2 - nby1, 0.0)
        nwlidx = (nwg - base.astype(jnp.float32)).astype(jnp.int32)
        return (nbx1, nby1, nbx2, nby2, nbarea, nwlidx)

    z = jnp.zeros((L,), jnp.float32)
    lax.fori_loop(0, NPRED, step_body,
                  (z, z, z, z, z, jnp.full((L,), -1, jnp.int32)))


@functools.partial(
    pl.kernel,
    out_type=jax.ShapeDtypeStruct((B, NPRED, 8, 128), jnp.float32),
    mesh=plsc.VectorSubcoreMesh(core_axis_name="c", subcore_axis_name="s"),
    compiler_params=pltpu.CompilerParams(
        needs_layout_passes=False, use_tc_tiling_on_sc=True),
    scratch_types=[
        pltpu.VMEM((CHUNK, C), jnp.float32),
        pltpu.VMEM((CHUNK, C), jnp.float32),
        pltpu.VMEM((14 * S,), jnp.float32),
        pltpu.VMEM((SPB * L,), jnp.float32),
        pltpu.VMEM((L,), jnp.float32),
        pltpu.VMEM((8, 128), jnp.float32),
        pltpu.VMEM((8, 128), jnp.float32),
        pltpu.VMEM((SPB, 8, 128), jnp.float32),
        pltpu.VMEM_SHARED((16, 8, 128), jnp.float32),
        pltpu.SemaphoreType.DMA,
        pltpu.SemaphoreType.DMA,
    ],
)
def _sc_nms(y, out, *scratch):
    _body(y, out, *scratch)


def kernel(y_pred):
    out = _sc_nms(y_pred)
    return out[:, :, 0, :13]


# speedup vs baseline: 1.5009x; 1.4541x over previous
import functools

import jax
import jax.numpy as jnp
from jax import lax
from jax.experimental import pallas as pl
from jax.experimental.pallas import tpu as pltpu
from jax.experimental.pallas import tpu_sc as plsc

INPUT_SIZE = 768.0
CONF_T = 0.01
IOU_T = 0.45
NPRED = 10

B = 8
N = 20000
C = 22
SPB = 4
NLOC = N // SPB
CHUNK = 200
NCHUNKS = NLOC // CHUNK
GPC = (CHUNK + 15) // 16
PAD = 5120
NGROUPS = PAD // 16
S = PAD
L = 16


def _splat_i(k):
    return jnp.full((L,), k, jnp.int32)


def _body(y, out, stage0, stage1, soa, g64, rec16, postbuf, recbuf, groupbuf,
          shared, sem0, sem1):
    c_id = lax.axis_index("c")
    s_id = lax.axis_index("s")
    b = c_id * SPB + s_id // SPB
    sl = s_id % SPB
    base = sl * NLOC
    leader = sl == 0
    iot = lax.iota(jnp.int32, L)

    def start_copy(c, stage, sem):
        pltpu.make_async_copy(
            y.at[b, pl.ds(base + c * CHUNK, CHUNK), :], stage, sem).start()

    def wait_copy(stage, sem):
        pltpu.make_async_copy(
            y.at[b, pl.ds(base, CHUNK), :], stage, sem).wait()

    def decode_chunk(c, stage):
        lch = c * CHUNK

        @plsc.parallel_loop(0, GPC)
        def dec_body(g):
            raw = g * L + iot
            valid = raw < CHUNK
            rows = jnp.minimum(raw, CHUNK - 1)

            def ld(k):
                return plsc.load_gather(stage, [rows, _splat_i(k)])

            off = lch + g * L
            idx0 = off + iot

            def st(r, v):
                plsc.store_scatter(soa, [r * S + idx0], v, mask=valid)

            sc = ld(1)
            sc = jnp.where(sc < CONF_T, -1.0, sc)
            pcx = ld(14)
            pcy = ld(15)
            pw = ld(16) + 1e-3
            ph = ld(17) + 1e-3
            va = ld(18) * pw
            vb = ld(19) * ph
            cx = ld(2) * va + pcx
            cy = ld(3) * vb + pcy
            w = jnp.exp(jnp.clip(ld(4) * ld(20), -10.0, 10.0)) * pw
            h = jnp.exp(jnp.clip(ld(5) * ld(21), -10.0, 10.0)) * ph
            x1 = (cx - 0.5 * w) * INPUT_SIZE
            y1 = (cy - 0.5 * h) * INPUT_SIZE
            x2 = (cx + 0.5 * w) * INPUT_SIZE
            y2 = (cy + 0.5 * h) * INPUT_SIZE
            ar = jnp.maximum(x2 - x1, 0.0) * jnp.maximum(y2 - y1, 0.0)
            st(0, sc)
            st(1, x1)
            st(2, y1)
            st(3, x2)
            st(4, y2)
            st(5, ar)
            for i in range(4):
                qx = (ld(6 + 2 * i) * va + pcx) * INPUT_SIZE
                qy = (ld(7 + 2 * i) * vb + pcy) * INPUT_SIZE
                st(6 + 2 * i, qx)
                st(7 + 2 * i, qy)

    start_copy(0, stage0, sem0)
    start_copy(1, stage1, sem1)

    def pair_body(p, carry):
        c0 = p * 2

        wait_copy(stage0, sem0)
        decode_chunk(c0, stage0)

        @pl.when(c0 + 2 < NCHUNKS)
        def _():
            start_copy(c0 + 2, stage0, sem0)

        wait_copy(stage1, sem1)
        decode_chunk(c0 + 1, stage1)

        @pl.when(c0 + 3 < NCHUNKS)
        def _():
            start_copy(c0 + 3, stage1, sem1)

        return carry

    lax.fori_loop(0, NCHUNKS // 2, pair_body, 0)
    if NCHUNKS % 2:
        wait_copy(stage0, sem0)
        decode_chunk(NCHUNKS - 1, stage0)

    pad0 = (NLOC // L) * L

    def pad_body(g, carry):
        idx0 = pad0 + g * L + iot
        m = idx0 >= NLOC
        plsc.store_scatter(soa, [idx0], jnp.full((L,), -1.0, jnp.float32),
                           mask=m)
        for r in range(1, 6):
            plsc.store_scatter(soa, [r * S + idx0], jnp.zeros((L,), jnp.float32),
                               mask=m)
        return carry

    lax.fori_loop(0, (PAD - pad0) // L, pad_body, 0)

    def step_body(step, carry):
        bx1, by1, bx2, by2, barea, wlidx = carry

        @plsc.parallel_loop(
            0, PAD, L, unroll=2,
            carry=(jnp.full((L,), -2.0, jnp.float32),
                   jnp.zeros((L,), jnp.int32)))
        def pass_result(o, vc):
            vbest, vbidx = vc
            sv = soa[pl.ds(o, L)]
            x1 = soa[pl.ds(S + o, L)]
            y1 = soa[pl.ds(2 * S + o, L)]
            x2 = soa[pl.ds(3 * S + o, L)]
            y2 = soa[pl.ds(4 * S + o, L)]
            ar = soa[pl.ds(5 * S + o, L)]
            iw = jnp.maximum(jnp.minimum(bx2, x2) - jnp.maximum(bx1, x1), 0.0)
            ih = jnp.maximum(jnp.minimum(by2, y2) - jnp.maximum(by1, y1), 0.0)
            inter = iw * ih
            iou = inter / (barea + ar - inter + 1e-8)
            lidx = o + iot
            supp = (iou > IOU_T) | (lidx == wlidx)
            s2 = jnp.where(supp, -1.0, sv)
            soa[pl.ds(o, L)] = s2
            upd = (s2 > vbest) | ((s2 == vbest) & (lidx < vbidx))
            vbest = jnp.where(upd, s2, vbest)
            vbidx = jnp.where(upd, lidx, vbidx)
            return (vbest, vbidx)

        vbest, vbidx = pass_result

        m = jnp.max(vbest)
        lidxm = jnp.where(vbest == m, vbidx, jnp.int32(2 ** 30))
        lidx = jnp.min(lidxm)
        gidxf = (base + lidx).astype(jnp.float32)

        rows = jnp.minimum(iot + jnp.where(iot >= 5, 1, 0), 13)
        cand = plsc.load_gather(soa, [rows * S + lidx])
        cand = jnp.where(iot == 13, gidxf, cand)
        cand = jnp.where(iot >= 14, 0.0, cand)
        postbuf[0, 0:L] = cand
        pltpu.sync_copy(postbuf, shared.at[s_id])
        plsc.subcore_barrier()
        pltpu.sync_copy(shared.at[pl.ds((s_id // SPB) * SPB, SPB), :, :],
                        groupbuf)
        plsc.subcore_barrier()
        for j in range(SPB):
            g64[pl.ds(j * L, L)] = groupbuf[j, 0, 0:L]

        rsel = jnp.minimum(iot, SPB - 1)
        sc4 = plsc.load_gather(g64, [rsel * L])
        id4 = plsc.load_gather(g64, [rsel * L + 13])
        lanem = iot < SPB
        sc4 = jnp.where(lanem, sc4, -3.0)
        m2 = jnp.max(sc4)
        sel2 = (sc4 == m2) & lanem
        wg = jnp.min(jnp.where(sel2, id4, 3e9))
        rowsel = jnp.where(sel2 & (id4 == wg), iot, jnp.int32(99))
        r = jnp.min(rowsel)
        rec = plsc.load_gather(g64, [_splat_i(0) + r * L + iot])
        rec16[...] = rec
        recbuf[0, 0:L] = rec

        @pl.when(leader)
        def _():
            pltpu.sync_copy(recbuf, out.at[b, step])

        nbx1 = plsc.load_gather(rec16, [_splat_i(1)])
        nby1 = plsc.load_gather(rec16, [_splat_i(2)])
        nbx2 = plsc.load_gather(rec16, [_splat_i(3)])
        nby2 = plsc.load_gather(rec16, [_splat_i(4)])
        nwg = plsc.load_gather(rec16, [_splat_i(13)])
        nbarea = jnp.maximum(nbx2 - nbx1, 0.0) * jnp.maximum(nby2 - nby1, 0.0)
        nwlidx = (nwg - base.astype(jnp.float32)).astype(jnp.int32)
        return (nbx1, nby1, nbx2, nby2, nbarea, nwlidx)

    z = jnp.zeros((L,), jnp.float32)
    lax.fori_loop(0, NPRED, step_body,
                  (z, z, z, z, z, jnp.full((L,), -1, jnp.int32)))


@functools.partial(
    pl.kernel,
    out_type=jax.ShapeDtypeStruct((B, NPRED, 8, 128), jnp.float32),
    mesh=plsc.VectorSubcoreMesh(core_axis_name="c", subcore_axis_name="s"),
    compiler_params=pltpu.CompilerParams(
        needs_layout_passes=False, use_tc_tiling_on_sc=True),
    scratch_types=[
        pltpu.VMEM((CHUNK, C), jnp.float32),
        pltpu.VMEM((CHUNK, C), jnp.float32),
        pltpu.VMEM((14 * S,), jnp.float32),
        pltpu.VMEM((SPB * L,), jnp.float32),
        pltpu.VMEM((L,), jnp.float32),
        pltpu.VMEM((8, 128), jnp.float32),
        pltpu.VMEM((8, 128), jnp.float32),
        pltpu.VMEM((SPB, 8, 128), jnp.float32),
        pltpu.VMEM_SHARED((16, 8, 128), jnp.float32),
        pltpu.SemaphoreType.DMA,
        pltpu.SemaphoreType.DMA,
    ],
)
def _sc_nms(y, out, *scratch):
    _body(y, out, *scratch)


def kernel(y_pred):
    out = _sc_nms(y_pred)
    return out[:, :, 0, :13]


# speedup vs baseline: 1.5010x; 1.0001x over previous
import functools

import jax
import jax.numpy as jnp
from jax import lax
from jax.experimental import pallas as pl
from jax.experimental.pallas import tpu as pltpu
from jax.experimental.pallas import tpu_sc as plsc

INPUT_SIZE = 768.0
CONF_T = 0.01
IOU_T = 0.45
NPRED = 10

B = 8
N = 20000
C = 22
SPB = 4
NLOC = N // SPB
CHUNK = 200
NCHUNKS = NLOC // CHUNK
GPC = (CHUNK + 15) // 16
PAD = 5120
NGROUPS = PAD // 16
S = PAD
L = 16


def _splat_i(k):
    return jnp.full((L,), k, jnp.int32)


def _body(y, out, stage0, stage1, soa, g64, rec16, postbuf, recbuf, groupbuf,
          shared, sem0, sem1):
    c_id = lax.axis_index("c")
    s_id = lax.axis_index("s")
    b = c_id * SPB + s_id // SPB
    sl = s_id % SPB
    base = sl * NLOC
    leader = sl == 0
    iot = lax.iota(jnp.int32, L)

    def start_copy(c, stage, sem):
        pltpu.make_async_copy(
            y.at[b, pl.ds(base + c * CHUNK, CHUNK), :], stage, sem).start()

    def wait_copy(stage, sem):
        pltpu.make_async_copy(
            y.at[b, pl.ds(base, CHUNK), :], stage, sem).wait()

    def decode_chunk(c, stage):
        lch = c * CHUNK

        @plsc.parallel_loop(0, GPC)
        def dec_body(g):
            raw = g * L + iot
            valid = raw < CHUNK
            rows = jnp.minimum(raw, CHUNK - 1)

            def ld(k):
                return plsc.load_gather(stage, [rows, _splat_i(k)])

            off = lch + g * L
            idx0 = off + iot

            def st(r, v):
                plsc.store_scatter(soa, [r * S + idx0], v, mask=valid)

            sc = ld(1)
            sc = jnp.where(sc < CONF_T, -1.0, sc)
            pcx = ld(14)
            pcy = ld(15)
            pw = ld(16) + 1e-3
            ph = ld(17) + 1e-3
            va = ld(18) * pw
            vb = ld(19) * ph
            cx = ld(2) * va + pcx
            cy = ld(3) * vb + pcy
            w = jnp.exp(jnp.clip(ld(4) * ld(20), -10.0, 10.0)) * pw
            h = jnp.exp(jnp.clip(ld(5) * ld(21), -10.0, 10.0)) * ph
            x1 = (cx - 0.5 * w) * INPUT_SIZE
            y1 = (cy - 0.5 * h) * INPUT_SIZE
            x2 = (cx + 0.5 * w) * INPUT_SIZE
            y2 = (cy + 0.5 * h) * INPUT_SIZE
            ar = jnp.maximum(x2 - x1, 0.0) * jnp.maximum(y2 - y1, 0.0)
            st(0, sc)
            st(1, x1)
            st(2, y1)
            st(3, x2)
            st(4, y2)
            st(5, ar)
            for i in range(4):
                qx = (ld(6 + 2 * i) * va + pcx) * INPUT_SIZE
                qy = (ld(7 + 2 * i) * vb + pcy) * INPUT_SIZE
                st(6 + 2 * i, qx)
                st(7 + 2 * i, qy)

    start_copy(0, stage0, sem0)
    start_copy(1, stage1, sem1)

    def pair_body(p, carry):
        c0 = p * 2

        wait_copy(stage0, sem0)
        decode_chunk(c0, stage0)

        @pl.when(c0 + 2 < NCHUNKS)
        def _():
            start_copy(c0 + 2, stage0, sem0)

        wait_copy(stage1, sem1)
        decode_chunk(c0 + 1, stage1)

        @pl.when(c0 + 3 < NCHUNKS)
        def _():
            start_copy(c0 + 3, stage1, sem1)

        return carry

    lax.fori_loop(0, NCHUNKS // 2, pair_body, 0)
    if NCHUNKS % 2:
        wait_copy(stage0, sem0)
        decode_chunk(NCHUNKS - 1, stage0)

    pad0 = (NLOC // L) * L

    def pad_body(g, carry):
        idx0 = pad0 + g * L + iot
        m = idx0 >= NLOC
        plsc.store_scatter(soa, [idx0], jnp.full((L,), -1.0, jnp.float32),
                           mask=m)
        for r in range(1, 6):
            plsc.store_scatter(soa, [r * S + idx0], jnp.zeros((L,), jnp.float32),
                               mask=m)
        return carry

    lax.fori_loop(0, (PAD - pad0) // L, pad_body, 0)

    def step_body(step, carry):
        bx1, by1, bx2, by2, barea, wlidx = carry

        @plsc.parallel_loop(
            0, PAD, L, unroll=3,
            carry=(jnp.full((L,), -2.0, jnp.float32),
                   jnp.zeros((L,), jnp.int32)))
        def pass_result(o, vc):
            vbest, vbidx = vc
            sv = soa[pl.ds(o, L)]
            x1 = soa[pl.ds(S + o, L)]
            y1 = soa[pl.ds(2 * S + o, L)]
            x2 = soa[pl.ds(3 * S + o, L)]
            y2 = soa[pl.ds(4 * S + o, L)]
            ar = soa[pl.ds(5 * S + o, L)]
            iw = jnp.maximum(jnp.minimum(bx2, x2) - jnp.maximum(bx1, x1), 0.0)
            ih = jnp.maximum(jnp.minimum(by2, y2) - jnp.maximum(by1, y1), 0.0)
            inter = iw * ih
            iou = inter / (barea + ar - inter + 1e-8)
            lidx = o + iot
            supp = (iou > IOU_T) | (lidx == wlidx)
            s2 = jnp.where(supp, -1.0, sv)
            soa[pl.ds(o, L)] = s2
            upd = (s2 > vbest) | ((s2 == vbest) & (lidx < vbidx))
            vbest = jnp.where(upd, s2, vbest)
            vbidx = jnp.where(upd, lidx, vbidx)
            return (vbest, vbidx)

        vbest, vbidx = pass_result

        m = jnp.max(vbest)
        lidxm = jnp.where(vbest == m, vbidx, jnp.int32(2 ** 30))
        lidx = jnp.min(lidxm)
        gidxf = (base + lidx).astype(jnp.float32)

        rows = jnp.minimum(iot + jnp.where(iot >= 5, 1, 0), 13)
        cand = plsc.load_gather(soa, [rows * S + lidx])
        cand = jnp.where(iot == 13, gidxf, cand)
        cand = jnp.where(iot >= 14, 0.0, cand)
        postbuf[0, 0:L] = cand
        pltpu.sync_copy(postbuf, shared.at[s_id])
        plsc.subcore_barrier()
        pltpu.sync_copy(shared.at[pl.ds((s_id // SPB) * SPB, SPB), :, :],
                        groupbuf)
        plsc.subcore_barrier()
        for j in range(SPB):
            g64[pl.ds(j * L, L)] = groupbuf[j, 0, 0:L]

        rsel = jnp.minimum(iot, SPB - 1)
        sc4 = plsc.load_gather(g64, [rsel * L])
        id4 = plsc.load_gather(g64, [rsel * L + 13])
        lanem = iot < SPB
        sc4 = jnp.where(lanem, sc4, -3.0)
        m2 = jnp.max(sc4)
        sel2 = (sc4 == m2) & lanem
        wg = jnp.min(jnp.where(sel2, id4, 3e9))
        rowsel = jnp.where(sel2 & (id4 == wg), iot, jnp.int32(99))
        r = jnp.min(rowsel)
        rec = plsc.load_gather(g64, [_splat_i(0) + r * L + iot])
        rec16[...] = rec
        recbuf[0, 0:L] = rec

        @pl.when(leader)
        def _():
            pltpu.sync_copy(recbuf, out.at[b, step])

        nbx1 = plsc.load_gather(rec16, [_splat_i(1)])
        nby1 = plsc.load_gather(rec16, [_splat_i(2)])
        nbx2 = plsc.load_gather(rec16, [_splat_i(3)])
        nby2 = plsc.load_gather(rec16, [_splat_i(4)])
        nwg = plsc.load_gather(rec16, [_splat_i(13)])
        nbarea = jnp.maximum(nbx2 - nbx1, 0.0) * jnp.maximum(nby2 - nby1, 0.0)
        nwlidx = (nwg - base.astype(jnp.float32)).astype(jnp.int32)
        return (nbx1, nby1, nbx2, nby2, nbarea, nwlidx)

    z = jnp.zeros((L,), jnp.float32)
    lax.fori_loop(0, NPRED, step_body,
                  (z, z, z, z, z, jnp.full((L,), -1, jnp.int32)))


@functools.partial(
    pl.kernel,
    out_type=jax.ShapeDtypeStruct((B, NPRED, 8, 128), jnp.float32),
    mesh=plsc.VectorSubcoreMesh(core_axis_name="c", subcore_axis_name="s"),
    compiler_params=pltpu.CompilerParams(
        needs_layout_passes=False, use_tc_tiling_on_sc=True),
    scratch_types=[
        pltpu.VMEM((CHUNK, C), jnp.float32),
        pltpu.VMEM((CHUNK, C), jnp.float32),
        pltpu.VMEM((14 * S,), jnp.float32),
        pltpu.VMEM((SPB * L,), jnp.float32),
        pltpu.VMEM((L,), jnp.float32),
        pltpu.VMEM((8, 128), jnp.float32),
        pltpu.VMEM((8, 128), jnp.float32),
        pltpu.VMEM((SPB, 8, 128), jnp.float32),
        pltpu.VMEM_SHARED((16, 8, 128), jnp.float32),
        pltpu.SemaphoreType.DMA,
        pltpu.SemaphoreType.DMA,
    ],
)
def _sc_nms(y, out, *scratch):
    _body(y, out, *scratch)


def kernel(y_pred):
    out = _sc_nms(y_pred)
    return out[:, :, 0, :13]


# speedup vs baseline: 1.5066x; 1.0037x over previous
import functools

import jax
import jax.numpy as jnp
from jax import lax
from jax.experimental import pallas as pl
from jax.experimental.pallas import tpu as pltpu
from jax.experimental.pallas import tpu_sc as plsc

INPUT_SIZE = 768.0
CONF_T = 0.01
IOU_T = 0.45
NPRED = 10

B = 8
N = 20000
C = 22
SPB = 4
NLOC = N // SPB
CHUNK = 200
NCHUNKS = NLOC // CHUNK
GPC = (CHUNK + 15) // 16
PAD = 5120
NGROUPS = PAD // 16
S = PAD
L = 16


def _splat_i(k):
    return jnp.full((L,), k, jnp.int32)


def _body(y, out, stage0, stage1, soa, g64, rec16, postbuf, recbuf, groupbuf,
          shared, sem0, sem1):
    c_id = lax.axis_index("c")
    s_id = lax.axis_index("s")
    b = c_id * SPB + s_id // SPB
    sl = s_id % SPB
    base = sl * NLOC
    leader = sl == 0
    iot = lax.iota(jnp.int32, L)

    def start_copy(c, stage, sem):
        pltpu.make_async_copy(
            y.at[b, pl.ds(base + c * CHUNK, CHUNK), :], stage, sem).start()

    def wait_copy(stage, sem):
        pltpu.make_async_copy(
            y.at[b, pl.ds(base, CHUNK), :], stage, sem).wait()

    def decode_chunk(c, stage):
        lch = c * CHUNK

        @plsc.parallel_loop(0, GPC)
        def dec_body(g):
            raw = g * L + iot
            valid = raw < CHUNK
            rows = jnp.minimum(raw, CHUNK - 1)

            def ld(k):
                return plsc.load_gather(stage, [rows, _splat_i(k)])

            off = lch + g * L
            idx0 = off + iot

            def st(r, v):
                plsc.store_scatter(soa, [r * S + idx0], v, mask=valid)

            sc = ld(1)
            sc = jnp.where(sc < CONF_T, -1.0, sc)
            pcx = ld(14)
            pcy = ld(15)
            pw = ld(16) + 1e-3
            ph = ld(17) + 1e-3
            va = ld(18) * pw
            vb = ld(19) * ph
            cx = ld(2) * va + pcx
            cy = ld(3) * vb + pcy
            w = jnp.exp(jnp.clip(ld(4) * ld(20), -10.0, 10.0)) * pw
            h = jnp.exp(jnp.clip(ld(5) * ld(21), -10.0, 10.0)) * ph
            x1 = (cx - 0.5 * w) * INPUT_SIZE
            y1 = (cy - 0.5 * h) * INPUT_SIZE
            x2 = (cx + 0.5 * w) * INPUT_SIZE
            y2 = (cy + 0.5 * h) * INPUT_SIZE
            ar = jnp.maximum(x2 - x1, 0.0) * jnp.maximum(y2 - y1, 0.0)
            st(0, sc)
            st(1, x1)
            st(2, y1)
            st(3, x2)
            st(4, y2)
            st(5, ar)
            for i in range(4):
                qx = (ld(6 + 2 * i) * va + pcx) * INPUT_SIZE
                qy = (ld(7 + 2 * i) * vb + pcy) * INPUT_SIZE
                st(6 + 2 * i, qx)
                st(7 + 2 * i, qy)

    start_copy(0, stage0, sem0)
    start_copy(1, stage1, sem1)

    def pair_body(p, carry):
        c0 = p * 2

        wait_copy(stage0, sem0)
        decode_chunk(c0, stage0)

        @pl.when(c0 + 2 < NCHUNKS)
        def _():
            start_copy(c0 + 2, stage0, sem0)

        wait_copy(stage1, sem1)
        decode_chunk(c0 + 1, stage1)

        @pl.when(c0 + 3 < NCHUNKS)
        def _():
            start_copy(c0 + 3, stage1, sem1)

        return carry

    lax.fori_loop(0, NCHUNKS // 2, pair_body, 0)
    if NCHUNKS % 2:
        wait_copy(stage0, sem0)
        decode_chunk(NCHUNKS - 1, stage0)

    pad0 = (NLOC // L) * L

    def pad_body(g, carry):
        idx0 = pad0 + g * L + iot
        m = idx0 >= NLOC
        plsc.store_scatter(soa, [idx0], jnp.full((L,), -1.0, jnp.float32),
                           mask=m)
        for r in range(1, 6):
            plsc.store_scatter(soa, [r * S + idx0], jnp.zeros((L,), jnp.float32),
                               mask=m)
        return carry

    lax.fori_loop(0, (PAD - pad0) // L, pad_body, 0)

    def step_body(step, carry):
        bx1, by1, bx2, by2, barea, wlidx = carry

        @plsc.parallel_loop(
            0, PAD, L, unroll=3,
            carry=(jnp.full((L,), -2.0, jnp.float32),
                   jnp.zeros((L,), jnp.int32)))
        def pass_result(o, vc):
            vbest, vbidx = vc
            sv = soa[pl.ds(o, L)]
            x1 = soa[pl.ds(S + o, L)]
            y1 = soa[pl.ds(2 * S + o, L)]
            x2 = soa[pl.ds(3 * S + o, L)]
            y2 = soa[pl.ds(4 * S + o, L)]
            ar = soa[pl.ds(5 * S + o, L)]
            iw = jnp.maximum(jnp.minimum(bx2, x2) - jnp.maximum(bx1, x1), 0.0)
            ih = jnp.maximum(jnp.minimum(by2, y2) - jnp.maximum(by1, y1), 0.0)
            inter = iw * ih
            iou = inter / (barea + ar - inter + 1e-8)
            lidx = o + iot
            supp = (iou > IOU_T) | (lidx == wlidx)
            s2 = jnp.where(supp, -1.0, sv)
            soa[pl.ds(o, L)] = s2
            upd = (s2 > vbest) | ((s2 == vbest) & (lidx < vbidx))
            vbest = jnp.where(upd, s2, vbest)
            vbidx = jnp.where(upd, lidx, vbidx)
            return (vbest, vbidx)

        vbest, vbidx = pass_result

        m = jnp.max(vbest)
        lidxm = jnp.where(vbest == m, vbidx, jnp.int32(2 ** 30))
        lidx = jnp.min(lidxm)
        gidxf = (base + lidx).astype(jnp.float32)

        rows = jnp.minimum(iot + jnp.where(iot >= 5, 1, 0), 13)
        cand = plsc.load_gather(soa, [rows * S + lidx])
        cand = jnp.where(iot == 13, gidxf, cand)
        cand = jnp.where(iot >= 14, 0.0, cand)
        postbuf[0, 0:L] = cand
        pltpu.sync_copy(postbuf, shared.at[s_id])
        plsc.subcore_barrier()
        pltpu.sync_copy(shared.at[pl.ds((s_id // SPB) * SPB, SPB), :, :],
                        groupbuf)
        for j in range(SPB):
            g64[pl.ds(j * L, L)] = groupbuf[j, 0, 0:L]

        rsel = jnp.minimum(iot, SPB - 1)
        sc4 = plsc.load_gather(g64, [rsel * L])
        id4 = plsc.load_gather(g64, [rsel * L + 13])
        lanem = iot < SPB
        sc4 = jnp.where(lanem, sc4, -3.0)
        m2 = jnp.max(sc4)
        sel2 = (sc4 == m2) & lanem
        wg = jnp.min(jnp.where(sel2, id4, 3e9))
        rowsel = jnp.where(sel2 & (id4 == wg), iot, jnp.int32(99))
        r = jnp.min(rowsel)
        rec = plsc.load_gather(g64, [_splat_i(0) + r * L + iot])
        rec16[...] = rec
        recbuf[0, 0:L] = rec

        @pl.when(leader)
        def _():
            pltpu.sync_copy(recbuf, out.at[b, step])

        nbx1 = plsc.load_gather(rec16, [_splat_i(1)])
        nby1 = plsc.load_gather(rec16, [_splat_i(2)])
        nbx2 = plsc.load_gather(rec16, [_splat_i(3)])
        nby2 = plsc.load_gather(rec16, [_splat_i(4)])
        nwg = plsc.load_gather(rec16, [_splat_i(13)])
        nbarea = jnp.maximum(nbx2 - nbx1, 0.0) * jnp.maximum(nby2 - nby1, 0.0)
        nwlidx = (nwg - base.astype(jnp.float32)).astype(jnp.int32)
        return (nbx1, nby1, nbx2, nby2, nbarea, nwlidx)

    z = jnp.zeros((L,), jnp.float32)
    lax.fori_loop(0, NPRED, step_body,
                  (z, z, z, z, z, jnp.full((L,), -1, jnp.int32)))


@functools.partial(
    pl.kernel,
    out_type=jax.ShapeDtypeStruct((B, NPRED, 8, 128), jnp.float32),
    mesh=plsc.VectorSubcoreMesh(core_axis_name="c", subcore_axis_name="s"),
    compiler_params=pltpu.CompilerParams(
        needs_layout_passes=False, use_tc_tiling_on_sc=True),
    scratch_types=[
        pltpu.VMEM((CHUNK, C), jnp.float32),
        pltpu.VMEM((CHUNK, C), jnp.float32),
        pltpu.VMEM((14 * S,), jnp.float32),
        pltpu.VMEM((SPB * L,), jnp.float32),
        pltpu.VMEM((L,), jnp.float32),
        pltpu.VMEM((8, 128), jnp.float32),
        pltpu.VMEM((8, 128), jnp.float32),
        pltpu.VMEM((SPB, 8, 128), jnp.float32),
        pltpu.VMEM_SHARED((16, 8, 128), jnp.float32),
        pltpu.SemaphoreType.DMA,
        pltpu.SemaphoreType.DMA,
    ],
)
def _sc_nms(y, out, *scratch):
    _body(y, out, *scratch)


def kernel(y_pred):
    out = _sc_nms(y_pred)
    return out[:, :, 0, :13]


# speedup vs baseline: 1.6350x; 1.0852x over previous
import functools

import jax
import jax.numpy as jnp
from jax import lax
from jax.experimental import pallas as pl
from jax.experimental.pallas import tpu as pltpu
from jax.experimental.pallas import tpu_sc as plsc

INPUT_SIZE = 768.0
CONF_T = 0.01
IOU_T = 0.45
NPRED = 10

B = 8
N = 20000
C = 22
SPB = 4
NLOC = N // SPB
CHUNK = 200
NCHUNKS = NLOC // CHUNK
GPC = (CHUNK + 15) // 16
PAD = 5120
NGROUPS = PAD // 16
S = PAD
L = 16


def _splat_i(k):
    return jnp.full((L,), k, jnp.int32)


def _body(y, out, stage0, stage1, soa, gmax, g64, rec16, postbuf, recbuf,
          groupbuf, shared, sem0, sem1):
    c_id = lax.axis_index("c")
    s_id = lax.axis_index("s")
    b = c_id * SPB + s_id // SPB
    sl = s_id % SPB
    base = sl * NLOC
    leader = sl == 0
    iot = lax.iota(jnp.int32, L)

    def start_copy(c, stage, sem):
        pltpu.make_async_copy(
            y.at[b, pl.ds(base + c * CHUNK, CHUNK), :], stage, sem).start()

    def wait_copy(stage, sem):
        pltpu.make_async_copy(
            y.at[b, pl.ds(base, CHUNK), :], stage, sem).wait()

    def decode_chunk(c, stage):
        lch = c * CHUNK

        @plsc.parallel_loop(0, GPC)
        def dec_body(g):
            raw = g * L + iot
            valid = raw < CHUNK
            rows = jnp.minimum(raw, CHUNK - 1)

            def ld(k):
                return plsc.load_gather(stage, [rows, _splat_i(k)])

            off = lch + g * L
            idx0 = off + iot

            def st(r, v):
                plsc.store_scatter(soa, [r * S + idx0], v, mask=valid)

            sc = ld(1)
            sc = jnp.where(sc < CONF_T, -1.0, sc)
            pcx = ld(14)
            pcy = ld(15)
            pw = ld(16) + 1e-3
            ph = ld(17) + 1e-3
            va = ld(18) * pw
            vb = ld(19) * ph
            cx = ld(2) * va + pcx
            cy = ld(3) * vb + pcy
            w = jnp.exp(jnp.clip(ld(4) * ld(20), -10.0, 10.0)) * pw
            h = jnp.exp(jnp.clip(ld(5) * ld(21), -10.0, 10.0)) * ph
            x1 = (cx - 0.5 * w) * INPUT_SIZE
            y1 = (cy - 0.5 * h) * INPUT_SIZE
            x2 = (cx + 0.5 * w) * INPUT_SIZE
            y2 = (cy + 0.5 * h) * INPUT_SIZE
            ar = jnp.maximum(x2 - x1, 0.0) * jnp.maximum(y2 - y1, 0.0)
            st(0, sc)
            st(1, x1)
            st(2, y1)
            st(3, x2)
            st(4, y2)
            st(5, ar)
            for i in range(4):
                qx = (ld(6 + 2 * i) * va + pcx) * INPUT_SIZE
                qy = (ld(7 + 2 * i) * vb + pcy) * INPUT_SIZE
                st(6 + 2 * i, qx)
                st(7 + 2 * i, qy)

    start_copy(0, stage0, sem0)
    start_copy(1, stage1, sem1)

    def pair_body(p, carry):
        c0 = p * 2

        wait_copy(stage0, sem0)
        decode_chunk(c0, stage0)

        @pl.when(c0 + 2 < NCHUNKS)
        def _():
            start_copy(c0 + 2, stage0, sem0)

        wait_copy(stage1, sem1)
        decode_chunk(c0 + 1, stage1)

        @pl.when(c0 + 3 < NCHUNKS)
        def _():
            start_copy(c0 + 3, stage1, sem1)

        return carry

    lax.fori_loop(0, NCHUNKS // 2, pair_body, 0)
    if NCHUNKS % 2:
        wait_copy(stage0, sem0)
        decode_chunk(NCHUNKS - 1, stage0)

    pad0 = (NLOC // L) * L

    def pad_body(g, carry):
        idx0 = pad0 + g * L + iot
        m = idx0 >= NLOC
        plsc.store_scatter(soa, [idx0], jnp.full((L,), -1.0, jnp.float32),
                           mask=m)
        for r in range(1, 6):
            plsc.store_scatter(soa, [r * S + idx0], jnp.zeros((L,), jnp.float32),
                               mask=m)
        return carry

    lax.fori_loop(0, (PAD - pad0) // L, pad_body, 0)

    def sum_body(g, carry):
        v = soa[pl.ds(g * L, L)]
        plsc.store_scatter(
            gmax, [jnp.zeros((L,), jnp.int32) + g],
            jnp.zeros((L,), jnp.float32) + jnp.max(v), mask=iot == 0)
        return carry

    lax.fori_loop(0, NGROUPS, sum_body, 0)

    def step_body(step, carry):
        wx1, wy1, wx2, wy2, warea = carry

        def scan_summary():
            def sc_body(j, vc):
                bv, bi = vc
                v = gmax[pl.ds(j * L, L)]
                gi = j * L + iot
                upd = (v > bv) | ((v == bv) & (gi < bi))
                return (jnp.where(upd, v, bv), jnp.where(upd, gi, bi))

            bv, bi = lax.fori_loop(
                0, NGROUPS // L, sc_body,
                (jnp.full((L,), -2.0, jnp.float32),
                 jnp.full((L,), 2 ** 30, jnp.int32)))
            m = jnp.max(bv)
            gsel = jnp.min(jnp.where(bv == m, bi, jnp.int32(2 ** 30)))
            return m, gsel

        def w_cond(st):
            return st[0] == 0

        def w_body(st):
            m, gsel = scan_summary()
            svec = soa[pl.ds(gsel * L, L)]
            lane = jnp.min(jnp.where(svec == m, iot, jnp.int32(99)))
            lidx = gsel * L + lane
            cx1 = plsc.load_gather(soa, [jnp.zeros((L,), jnp.int32) + S + lidx])
            cy1 = plsc.load_gather(soa,
                                   [jnp.zeros((L,), jnp.int32) + 2 * S + lidx])
            cx2 = plsc.load_gather(soa,
                                   [jnp.zeros((L,), jnp.int32) + 3 * S + lidx])
            cy2 = plsc.load_gather(soa,
                                   [jnp.zeros((L,), jnp.int32) + 4 * S + lidx])
            car = plsc.load_gather(soa,
                                   [jnp.zeros((L,), jnp.int32) + 5 * S + lidx])
            iw = jnp.maximum(jnp.minimum(wx2, cx2) - jnp.maximum(wx1, cx1),
                             0.0)
            ih = jnp.maximum(jnp.minimum(wy2, cy2) - jnp.maximum(wy1, cy1),
                             0.0)
            inter = iw * ih
            iou = inter / (warea + car - inter + 1e-8)
            bad = jnp.max(iou) > IOU_T
            ok = (m < 0.0) | jnp.logical_not(bad)

            @pl.when(jnp.logical_not(ok))
            def _():
                plsc.store_scatter(
                    soa, [jnp.zeros((L,), jnp.int32) + lidx],
                    jnp.full((L,), -1.0, jnp.float32), mask=iot == 0)
                nv = jnp.where(iot == lane, -1.0, svec)
                plsc.store_scatter(
                    gmax, [jnp.zeros((L,), jnp.int32) + gsel],
                    jnp.zeros((L,), jnp.float32) + jnp.max(nv), mask=iot == 0)

            return (jnp.where(ok, jnp.int32(1), jnp.int32(0)), lidx, m)

        _, lidx, mval = lax.while_loop(
            w_cond, w_body, (jnp.int32(0), jnp.int32(0), jnp.float32(-2.0)))
        gidxf = (base + lidx).astype(jnp.float32)

        rows = jnp.minimum(iot + jnp.where(iot >= 5, 1, 0), 13)
        cand = plsc.load_gather(soa, [rows * S + lidx])
        cand = jnp.where(iot == 13, gidxf, cand)
        cand = jnp.where(iot >= 14, 0.0, cand)
        postbuf[0, 0:L] = cand
        pltpu.sync_copy(postbuf, shared.at[s_id])
        plsc.subcore_barrier()
        pltpu.sync_copy(shared.at[pl.ds((s_id // SPB) * SPB, SPB), :, :],
                        groupbuf)
        for j in range(SPB):
            g64[pl.ds(j * L, L)] = groupbuf[j, 0, 0:L]

        rsel = jnp.minimum(iot, SPB - 1)
        sc4 = plsc.load_gather(g64, [rsel * L])
        id4 = plsc.load_gather(g64, [rsel * L + 13])
        lanem = iot < SPB
        sc4 = jnp.where(lanem, sc4, -3.0)
        m2 = jnp.max(sc4)
        sel2 = (sc4 == m2) & lanem
        wg = jnp.min(jnp.where(sel2, id4, 3e9))
        rowsel = jnp.where(sel2 & (id4 == wg), iot, jnp.int32(99))
        r = jnp.min(rowsel)
        rec = plsc.load_gather(g64, [_splat_i(0) + r * L + iot])
        rec16[...] = rec
        recbuf[0, 0:L] = rec

        @pl.when(leader)
        def _():
            pltpu.sync_copy(recbuf, out.at[b, step])

        nbx1 = plsc.load_gather(rec16, [_splat_i(1)])
        nby1 = plsc.load_gather(rec16, [_splat_i(2)])
        nbx2 = plsc.load_gather(rec16, [_splat_i(3)])
        nby2 = plsc.load_gather(rec16, [_splat_i(4)])
        nwg = plsc.load_gather(rec16, [_splat_i(13)])
        nbarea = jnp.maximum(nbx2 - nbx1, 0.0) * jnp.maximum(nby2 - nby1, 0.0)

        ins = iot == step
        wx1 = jnp.where(ins, nbx1, wx1)
        wy1 = jnp.where(ins, nby1, wy1)
        wx2 = jnp.where(ins, nbx2, wx2)
        wy2 = jnp.where(ins, nby2, wy2)
        warea = jnp.where(ins, nbarea, warea)

        wls = jnp.max((nwg - base.astype(jnp.float32)).astype(jnp.int32))
        own = (wls >= 0) & (wls < NLOC)

        @pl.when(own)
        def _():
            gw = wls // L
            wlane = wls - gw * L
            gvec = soa[pl.ds(gw * L, L)]
            plsc.store_scatter(
                soa, [jnp.zeros((L,), jnp.int32) + wls],
                jnp.full((L,), -1.0, jnp.float32), mask=iot == 0)
            nv = jnp.where(iot == wlane, -1.0, gvec)
            plsc.store_scatter(
                gmax, [jnp.zeros((L,), jnp.int32) + gw],
                jnp.zeros((L,), jnp.float32) + jnp.max(nv), mask=iot == 0)

        return (wx1, wy1, wx2, wy2, warea)

    z = jnp.zeros((L,), jnp.float32)
    lax.fori_loop(0, NPRED, step_body, (z, z, z, z, z))


@functools.partial(
    pl.kernel,
    out_type=jax.ShapeDtypeStruct((B, NPRED, 8, 128), jnp.float32),
    mesh=plsc.VectorSubcoreMesh(core_axis_name="c", subcore_axis_name="s"),
    compiler_params=pltpu.CompilerParams(
        needs_layout_passes=False, use_tc_tiling_on_sc=True),
    scratch_types=[
        pltpu.VMEM((CHUNK, C), jnp.float32),
        pltpu.VMEM((CHUNK, C), jnp.float32),
        pltpu.VMEM((14 * S,), jnp.float32),
        pltpu.VMEM((NGROUPS,), jnp.float32),
        pltpu.VMEM((SPB * L,), jnp.float32),
        pltpu.VMEM((L,), jnp.float32),
        pltpu.VMEM((8, 128), jnp.float32),
        pltpu.VMEM((8, 128), jnp.float32),
        pltpu.VMEM((SPB, 8, 128), jnp.float32),
        pltpu.VMEM_SHARED((16, 8, 128), jnp.float32),
        pltpu.SemaphoreType.DMA,
        pltpu.SemaphoreType.DMA,
    ],
)
def _sc_nms(y, out, *scratch):
    _body(y, out, *scratch)


def kernel(y_pred):
    out = _sc_nms(y_pred)
    return out[:, :, 0, :13]
